# Initial kernel scaffold; baseline (speedup 1.0000x reference)
import jax
import jax.numpy as jnp
from jax.experimental import pallas as pl


def kernel(user_genre_contexts, user_tag_contexts, user_watch_history, user_watch_history_ratings, timestamps, movie_genres, movie_tags, movie_genome_tags, years, target_movieId, item_table, W_item, b_item, W_ig, b_ig, W_it, b_it, W_igt, b_igt, year_table, W_y, b_y, W_ug, b_ug, ts_table, W_ts, b_ts):
    raise NotImplementedError("write your pallas kernel here")


# speedup vs baseline: 12.4274x; 12.4274x over previous
import functools

import jax
import jax.numpy as jnp
from jax import lax
from jax.experimental import pallas as pl
from jax.experimental.pallas import tpu as pltpu
from jax.experimental.pallas import tpu_sc as plsc

B = 4096
HIST = 200
D = 40
DP = 48
NC = 2
NS = 16
NW = NC * NS
BPW = B // NW
SPL = 104
SPL2 = HIST - SPL


def _sc_pool(table, idx_flat, rat_flat, tidx, pool_out, trows_out,
             idx_v, rat_v, w_v, tidx_v, rows_a, rows_b, out_v, trows_v,
             sem_a, sem_b, sem_t):
    pad_idx = table.shape[0] - 1
    wid = lax.axis_index("s") * NC + lax.axis_index("c")
    base = wid * BPW

    pltpu.sync_copy(idx_flat.at[pl.ds(base * HIST, BPW * HIST)], idx_v)
    pltpu.sync_copy(rat_flat.at[pl.ds(base * HIST, BPW * HIST)], rat_v)
    pltpu.sync_copy(tidx.at[pl.ds(base, BPW)], tidx_v)
    pltpu.async_copy(table.at[tidx_v], trows_v, sem_t)

    def w_body(i, _):
        iv = idx_v[pl.ds(i * 16, 16)]
        rv = rat_v[pl.ds(i * 16, 16)]
        w_v[pl.ds(i * 16, 16)] = jnp.where(iv == pad_idx, 0.0, jnp.abs(rv))
        return 0
    lax.fori_loop(0, (BPW * HIST) // 16, w_body, 0)

    def issue(s, buf, sem):
        off = s * HIST
        pltpu.async_copy(table.at[idx_v.at[pl.ds(off, SPL)]],
                         buf.at[pl.ds(0, SPL)], sem)
        pltpu.async_copy(table.at[idx_v.at[pl.ds(off + SPL, SPL2)]],
                         buf.at[pl.ds(SPL, SPL2)], sem)

    def wait(s, buf, sem):
        off = s * HIST
        pltpu.make_async_copy(table.at[idx_v.at[pl.ds(off, SPL)]],
                              buf.at[pl.ds(0, SPL)], sem).wait()
        pltpu.make_async_copy(table.at[idx_v.at[pl.ds(off + SPL, SPL2)]],
                              buf.at[pl.ds(SPL, SPL2)], sem).wait()

    lane = lax.iota(jnp.int32, 16)
    hi8 = lane >= 8
    zero16 = jnp.zeros((16,), jnp.float32)

    def compute(s, buf):
        off = s * HIST

        def rows16(a0, a1, a2, wsum, wv, g, n):
            for k in range(n):
                h = g * 16 + k
                w = wv[k]
                wsum[k % 4] = wsum[k % 4] + w
                r0 = buf[h, pl.ds(0, 16)]
                r1 = buf[h, pl.ds(16, 16)]
                r2 = buf[h, pl.ds(24, 16)]
                w2 = jnp.where(hi8, w, 0.0)
                a0 = a0 + w * r0
                a1 = a1 + w * r1
                a2 = a2 + w2 * r2
            return a0, a1, a2

        def grp_body(g, carry):
            a0, a1, a2, s0, s1, s2, s3 = carry
            wsum = [s0, s1, s2, s3]
            wv = w_v[pl.ds(off + g * 16, 16)]
            a0, a1, a2 = rows16(a0, a1, a2, wsum, wv, g, 16)
            return (a0, a1, a2, *wsum)

        z = jnp.float32(0.0)
        a0, a1, a2, s0, s1, s2, s3 = lax.fori_loop(
            0, (HIST // 16), grp_body, (zero16, zero16, zero16, z, z, z, z))
        wsum = [s0, s1, s2, s3]
        wv = w_v[pl.ds(off + 192, 16)]
        a0, a1, a2 = rows16(a0, a1, a2, wsum, wv, 12, 8)
        ws = (wsum[0] + wsum[1]) + (wsum[2] + wsum[3])
        wsb = jnp.broadcast_to(ws, (16,))
        inv = 1.0 / jnp.maximum(wsb, 1e-6)
        ob = s * DP
        out_v[pl.ds(ob + 24, 16)] = a2 * inv
        out_v[pl.ds(ob + 16, 16)] = a1 * inv
        out_v[pl.ds(ob, 16)] = a0 * inv
        return ws

    issue(0, rows_a, sem_a)

    def chunk(g, _):
        s = g * 2
        issue(s + 1, rows_b, sem_b)
        wait(s, rows_a, sem_a)
        compute(s, rows_a)
        s2 = jnp.minimum(s + 2, BPW - 1)
        issue(s2, rows_a, sem_a)
        wait(s + 1, rows_b, sem_b)
        compute(s + 1, rows_b)
        return _

    lax.fori_loop(0, BPW // 2, chunk, 0)
    wait(BPW - 1, rows_a, sem_a)
    pltpu.make_async_copy(table.at[tidx_v], trows_v, sem_t).wait()

    pltpu.sync_copy(out_v, pool_out.at[pl.ds(base * DP, BPW * DP)])
    pltpu.sync_copy(trows_v, trows_out.at[pl.ds(base, BPW)])


@functools.lru_cache(maxsize=1)
def _sc_pool_call():
    return pl.kernel(
        _sc_pool,
        out_type=(
            jax.ShapeDtypeStruct((B * DP,), jnp.float32),
            jax.ShapeDtypeStruct((B, D), jnp.float32),
        ),
        mesh=plsc.VectorSubcoreMesh(
            core_axis_name="c", subcore_axis_name="s",
            num_cores=NC, num_subcores=NS),
        compiler_params=pltpu.CompilerParams(use_tc_tiling_on_sc=False),
        scratch_types=[
        pltpu.VMEM((BPW * HIST,), jnp.int32),
        pltpu.VMEM((BPW * HIST,), jnp.float32),
        pltpu.VMEM((BPW * HIST + 16,), jnp.float32),
        pltpu.VMEM((BPW,), jnp.int32),
        pltpu.VMEM((HIST, D), jnp.float32),
        pltpu.VMEM((HIST, D), jnp.float32),
        pltpu.VMEM((BPW * DP,), jnp.float32),
        pltpu.VMEM((BPW, D), jnp.float32),
            pltpu.SemaphoreType.DMA,
            pltpu.SemaphoreType.DMA,
            pltpu.SemaphoreType.DMA,
        ],
    )


BLK = 512
NBLK = B // BLK
YEARS = 120
TSB = 1000


def _tc_dense(genres_ref, tags_ref, genome_ref, ugc_ref, y_ref, ts_ref,
              hist_ref, trow_ref, w_ig, b_ig, w_it, b_it, w_igt, b_igt,
              w_item, b_item, ytab, w_y, b_y, ttab, w_ts, b_ts, w_ug, b_ug,
              out_ref):
    f32 = jnp.float32

    def mm(a, b):
        return jnp.dot(a, b, preferred_element_type=f32)

    ig = jnp.tanh(mm(genres_ref[...], w_ig[...]) + b_ig[...])
    it = jnp.tanh(mm(tags_ref[...], w_it[...]) + b_it[...])
    igt = jnp.tanh(mm(genome_ref[...], w_igt[...]) + b_igt[...])
    item = jnp.tanh(mm(trow_ref[...], w_item[...]) + b_item[...])

    y_col = y_ref[...]
    oh_y = (y_col == lax.broadcasted_iota(jnp.int32, (BLK, YEARS), 1)
            .astype(f32)).astype(f32)
    yemb = jnp.tanh(mm(oh_y, mm(ytab[...], w_y[...])) + b_y[...])

    ts_col = ts_ref[...]
    oh_ts = (ts_col == lax.broadcasted_iota(jnp.int32, (BLK, TSB), 1)
             .astype(f32)).astype(f32)
    tsemb = jnp.tanh(mm(oh_ts, mm(ttab[...], w_ts[...])) + b_ts[...])

    genre = jnp.tanh(mm(ugc_ref[...], w_ug[...]) + b_ug[...])
    hist = hist_ref[:, :D]

    u = jnp.concatenate([hist, genre, tsemb], axis=1)
    v = jnp.concatenate([ig, it, igt, item, yemb], axis=1)
    out_ref[...] = jnp.sum(u * v, axis=1, keepdims=True)


def _row_spec(cols):
    return pl.BlockSpec((BLK, cols), lambda i: (i, 0))


def _full_spec(shape):
    nd = len(shape)
    return pl.BlockSpec(shape, lambda i: (0,) * nd)


def kernel(user_genre_contexts, user_tag_contexts, user_watch_history,
           user_watch_history_ratings, timestamps, movie_genres, movie_tags,
           movie_genome_tags, years, target_movieId, item_table, W_item,
           b_item, W_ig, b_ig, W_it, b_it, W_igt, b_igt, year_table, W_y,
           b_y, W_ug, b_ug, ts_table, W_ts, b_ts):
    idx_flat = user_watch_history.astype(jnp.int32).reshape(-1)
    rat_flat = user_watch_history_ratings.reshape(-1)
    tidx = target_movieId.astype(jnp.int32)

    pool_flat, trows = _sc_pool_call()(item_table, idx_flat, rat_flat, tidx)
    hist_pool = pool_flat.reshape(B, DP)

    y_col = years.astype(jnp.float32).reshape(B, 1)
    ts_col = timestamps.astype(jnp.float32).reshape(B, 1)
    b2 = lambda x: x.reshape(1, -1)

    out = pl.pallas_call(
        _tc_dense,
        grid=(NBLK,),
        in_specs=[
            _row_spec(movie_genres.shape[1]),
            _row_spec(movie_tags.shape[1]),
            _row_spec(movie_genome_tags.shape[1]),
            _row_spec(user_genre_contexts.shape[1]),
            _row_spec(1),
            _row_spec(1),
            _row_spec(DP),
            _row_spec(D),
            _full_spec(W_ig.shape), _full_spec((1, b_ig.shape[0])),
            _full_spec(W_it.shape), _full_spec((1, b_it.shape[0])),
            _full_spec(W_igt.shape), _full_spec((1, b_igt.shape[0])),
            _full_spec(W_item.shape), _full_spec((1, b_item.shape[0])),
            _full_spec(year_table.shape), _full_spec(W_y.shape),
            _full_spec((1, b_y.shape[0])),
            _full_spec(ts_table.shape), _full_spec(W_ts.shape),
            _full_spec((1, b_ts.shape[0])),
            _full_spec(W_ug.shape), _full_spec((1, b_ug.shape[0])),
        ],
        out_specs=pl.BlockSpec((BLK, 1), lambda i: (i, 0)),
        out_shape=jax.ShapeDtypeStruct((B, 1), jnp.float32),
    )(movie_genres, movie_tags, movie_genome_tags, user_genre_contexts,
      y_col, ts_col, hist_pool, trows,
      W_ig, b2(b_ig), W_it, b2(b_it), W_igt, b2(b_igt), W_item, b2(b_item),
      year_table, W_y, b2(b_y), ts_table, W_ts, b2(b_ts), W_ug, b2(b_ug))

    return out.reshape(B)


# speedup vs baseline: 12.5587x; 1.0106x over previous
import functools

import jax
import jax.numpy as jnp
from jax import lax
from jax.experimental import pallas as pl
from jax.experimental.pallas import tpu as pltpu
from jax.experimental.pallas import tpu_sc as plsc

B = 4096
HIST = 200
D = 40
DP = 48
NC = 2
NS = 16
NW = NC * NS
BPW = B // NW
SPL = 104
SPL2 = HIST - SPL


def _sc_pool(table, idx2, rat2, tidx, pool_out, trows_out,
             idx_v, rat_v, w_v, tidx_v, rows_a, rows_b, out_v, trows_v,
             sem_a, sem_b, sem_t):
    pad_idx = table.shape[0] - 1
    wid = lax.axis_index("s") * NC + lax.axis_index("c")
    base = wid * BPW

    pltpu.sync_copy(idx2.at[pl.ds(base, BPW)], idx_v)
    pltpu.sync_copy(rat2.at[pl.ds(base, BPW)], rat_v)
    pltpu.sync_copy(tidx.at[pl.ds(base, BPW)], tidx_v)
    pltpu.async_copy(table.at[tidx_v], trows_v, sem_t)

    chunk_offs = tuple(range(0, HIST - 16, 16)) + (HIST - 16,)

    def w_body(s, _):
        for c in chunk_offs:
            iv = idx_v[s, pl.ds(c, 16)]
            rv = rat_v[s, pl.ds(c, 16)]
            w_v[s, pl.ds(c, 16)] = jnp.where(iv == pad_idx, 0.0,
                                             jnp.abs(rv))
        return 0
    lax.fori_loop(0, BPW, w_body, 0)

    def issue(s, buf, sem):
        pltpu.async_copy(table.at[idx_v.at[s, pl.ds(0, SPL)]],
                         buf.at[pl.ds(0, SPL)], sem)
        pltpu.async_copy(table.at[idx_v.at[s, pl.ds(SPL, SPL2)]],
                         buf.at[pl.ds(SPL, SPL2)], sem)

    def wait(s, buf, sem):
        pltpu.make_async_copy(table.at[idx_v.at[s, pl.ds(0, SPL)]],
                              buf.at[pl.ds(0, SPL)], sem).wait()
        pltpu.make_async_copy(table.at[idx_v.at[s, pl.ds(SPL, SPL2)]],
                              buf.at[pl.ds(SPL, SPL2)], sem).wait()

    zero16 = jnp.zeros((16,), jnp.float32)

    def compute(s, buf):
        def rows16(a0, a1, a2, wsum, wv, g, n, lane_off=0):
            for k in range(n):
                h = g * 16 + k
                w = wv[k + lane_off]
                wsum[k % 4] = wsum[k % 4] + w
                r0 = buf[h, pl.ds(0, 16)]
                r1 = buf[h, pl.ds(16, 16)]
                r2 = buf[h, pl.ds(24, 16)]
                a0 = a0 + w * r0
                a1 = a1 + w * r1
                a2 = a2 + w * r2
            return a0, a1, a2

        def grp_body(g, carry):
            a0, a1, a2, s0, s1, s2, s3 = carry
            wsum = [s0, s1, s2, s3]
            wv = w_v[s, pl.ds(g * 16, 16)]
            a0, a1, a2 = rows16(a0, a1, a2, wsum, wv, g, 16)
            return (a0, a1, a2, *wsum)

        z = jnp.float32(0.0)
        a0, a1, a2, s0, s1, s2, s3 = lax.fori_loop(
            0, (HIST // 16), grp_body, (zero16, zero16, zero16, z, z, z, z))
        wsum = [s0, s1, s2, s3]
        wv = w_v[s, pl.ds(HIST - 16, 16)]
        a0, a1, a2 = rows16(a0, a1, a2, wsum, wv, 12, 8, lane_off=8)
        ws = (wsum[0] + wsum[1]) + (wsum[2] + wsum[3])
        wsb = jnp.broadcast_to(ws, (16,))
        inv = 1.0 / jnp.maximum(wsb, 1e-6)
        ob = s * DP
        out_v[pl.ds(ob + 24, 16)] = a2 * inv
        out_v[pl.ds(ob + 16, 16)] = a1 * inv
        out_v[pl.ds(ob, 16)] = a0 * inv
        return ws

    issue(0, rows_a, sem_a)

    def chunk(g, _):
        s = g * 2
        issue(s + 1, rows_b, sem_b)
        wait(s, rows_a, sem_a)
        compute(s, rows_a)
        s2 = jnp.minimum(s + 2, BPW - 1)
        issue(s2, rows_a, sem_a)
        wait(s + 1, rows_b, sem_b)
        compute(s + 1, rows_b)
        return _

    lax.fori_loop(0, BPW // 2, chunk, 0)
    wait(BPW - 1, rows_a, sem_a)
    pltpu.make_async_copy(table.at[tidx_v], trows_v, sem_t).wait()

    pltpu.sync_copy(out_v, pool_out.at[pl.ds(base * DP, BPW * DP)])
    pltpu.sync_copy(trows_v, trows_out.at[pl.ds(base, BPW)])


@functools.lru_cache(maxsize=1)
def _sc_pool_call():
    return pl.kernel(
        _sc_pool,
        out_type=(
            jax.ShapeDtypeStruct((B * DP,), jnp.float32),
            jax.ShapeDtypeStruct((B, D), jnp.float32),
        ),
        mesh=plsc.VectorSubcoreMesh(
            core_axis_name="c", subcore_axis_name="s",
            num_cores=NC, num_subcores=NS),
        compiler_params=pltpu.CompilerParams(use_tc_tiling_on_sc=False),
        scratch_types=[
        pltpu.VMEM((BPW, HIST), jnp.int32),
        pltpu.VMEM((BPW, HIST), jnp.float32),
        pltpu.VMEM((BPW, HIST), jnp.float32),
        pltpu.VMEM((BPW,), jnp.int32),
        pltpu.VMEM((HIST, D), jnp.float32),
        pltpu.VMEM((HIST, D), jnp.float32),
        pltpu.VMEM((BPW * DP,), jnp.float32),
        pltpu.VMEM((BPW, D), jnp.float32),
            pltpu.SemaphoreType.DMA,
            pltpu.SemaphoreType.DMA,
            pltpu.SemaphoreType.DMA,
        ],
    )


BLK = 512
NBLK = B // BLK
YEARS = 120
TSB = 1000


def _tc_dense(genres_ref, tags_ref, genome_ref, ugc_ref, y_ref, ts_ref,
              hist_ref, trow_ref, w_ig, b_ig, w_it, b_it, w_igt, b_igt,
              w_item, b_item, ytab, w_y, b_y, ttab, w_ts, b_ts, w_ug, b_ug,
              out_ref):
    f32 = jnp.float32

    def mm(a, b):
        return jnp.dot(a, b, preferred_element_type=f32)

    ig = jnp.tanh(mm(genres_ref[...], w_ig[...]) + b_ig[...])
    it = jnp.tanh(mm(tags_ref[...], w_it[...]) + b_it[...])
    igt = jnp.tanh(mm(genome_ref[...], w_igt[...]) + b_igt[...])
    item = jnp.tanh(mm(trow_ref[...], w_item[...]) + b_item[...])

    y_col = y_ref[...]
    oh_y = (y_col == lax.broadcasted_iota(jnp.int32, (BLK, YEARS), 1)
            .astype(f32)).astype(f32)
    yemb = jnp.tanh(mm(oh_y, mm(ytab[...], w_y[...])) + b_y[...])

    ts_col = ts_ref[...]
    oh_ts = (ts_col == lax.broadcasted_iota(jnp.int32, (BLK, TSB), 1)
             .astype(f32)).astype(f32)
    tsemb = jnp.tanh(mm(oh_ts, mm(ttab[...], w_ts[...])) + b_ts[...])

    genre = jnp.tanh(mm(ugc_ref[...], w_ug[...]) + b_ug[...])
    hist = hist_ref[:, :D]

    u = jnp.concatenate([hist, genre, tsemb], axis=1)
    v = jnp.concatenate([ig, it, igt, item, yemb], axis=1)
    out_ref[...] = jnp.sum(u * v, axis=1, keepdims=True)


def _row_spec(cols):
    return pl.BlockSpec((BLK, cols), lambda i: (i, 0))


def _full_spec(shape):
    nd = len(shape)
    return pl.BlockSpec(shape, lambda i: (0,) * nd)


def kernel(user_genre_contexts, user_tag_contexts, user_watch_history,
           user_watch_history_ratings, timestamps, movie_genres, movie_tags,
           movie_genome_tags, years, target_movieId, item_table, W_item,
           b_item, W_ig, b_ig, W_it, b_it, W_igt, b_igt, year_table, W_y,
           b_y, W_ug, b_ug, ts_table, W_ts, b_ts):
    idx2 = user_watch_history.astype(jnp.int32)
    tidx = target_movieId.astype(jnp.int32)

    pool_flat, trows = _sc_pool_call()(
        item_table, idx2, user_watch_history_ratings, tidx)
    hist_pool = pool_flat.reshape(B, DP)

    y_col = years.astype(jnp.float32).reshape(B, 1)
    ts_col = timestamps.astype(jnp.float32).reshape(B, 1)
    b2 = lambda x: x.reshape(1, -1)

    out = pl.pallas_call(
        _tc_dense,
        grid=(NBLK,),
        in_specs=[
            _row_spec(movie_genres.shape[1]),
            _row_spec(movie_tags.shape[1]),
            _row_spec(movie_genome_tags.shape[1]),
            _row_spec(user_genre_contexts.shape[1]),
            _row_spec(1),
            _row_spec(1),
            _row_spec(DP),
            _row_spec(D),
            _full_spec(W_ig.shape), _full_spec((1, b_ig.shape[0])),
            _full_spec(W_it.shape), _full_spec((1, b_it.shape[0])),
            _full_spec(W_igt.shape), _full_spec((1, b_igt.shape[0])),
            _full_spec(W_item.shape), _full_spec((1, b_item.shape[0])),
            _full_spec(year_table.shape), _full_spec(W_y.shape),
            _full_spec((1, b_y.shape[0])),
            _full_spec(ts_table.shape), _full_spec(W_ts.shape),
            _full_spec((1, b_ts.shape[0])),
            _full_spec(W_ug.shape), _full_spec((1, b_ug.shape[0])),
        ],
        out_specs=pl.BlockSpec((BLK, 1), lambda i: (i, 0)),
        out_shape=jax.ShapeDtypeStruct((B, 1), jnp.float32),
    )(movie_genres, movie_tags, movie_genome_tags, user_genre_contexts,
      y_col, ts_col, hist_pool, trows,
      W_ig, b2(b_ig), W_it, b2(b_it), W_igt, b2(b_igt), W_item, b2(b_item),
      year_table, W_y, b2(b_y), ts_table, W_ts, b2(b_ts), W_ug, b2(b_ug))

    return out.reshape(B)


# speedup vs baseline: 12.7440x; 1.0148x over previous
import functools

import jax
import jax.numpy as jnp
from jax import lax
from jax.experimental import pallas as pl
from jax.experimental.pallas import tpu as pltpu
from jax.experimental.pallas import tpu_sc as plsc

B = 4096
HIST = 200
D = 40
DP = 48
TP = 16
NC = 2
NS = 16
NW = NC * NS
BPW = B // NW
SPL = 104
SPL2 = HIST - SPL


def _sc_pool(table, idx2, rat2, tidx, ytab, tstab, yidx, tsidx,
             pool_out, trows_out, yrows_out, tsrows_out,
             idx_v, rat_v, w_v, tidx_v, yidx_v, tsidx_v,
             rows_a, rows_b, out_v, trows_v, yrows_v, tsrows_v,
             sem_a, sem_b, sem_t, sem_y, sem_ts):
    pad_idx = table.shape[0] - 1
    wid = lax.axis_index("s") * NC + lax.axis_index("c")
    base = wid * BPW

    pltpu.sync_copy(idx2.at[pl.ds(base, BPW)], idx_v)
    pltpu.sync_copy(rat2.at[pl.ds(base, BPW)], rat_v)
    pltpu.sync_copy(tidx.at[pl.ds(base, BPW)], tidx_v)
    pltpu.sync_copy(yidx.at[pl.ds(base, BPW)], yidx_v)
    pltpu.sync_copy(tsidx.at[pl.ds(base, BPW)], tsidx_v)
    pltpu.async_copy(table.at[tidx_v], trows_v, sem_t)
    pltpu.async_copy(ytab.at[yidx_v], yrows_v, sem_y)
    pltpu.async_copy(tstab.at[tsidx_v], tsrows_v, sem_ts)

    chunk_offs = tuple(range(0, HIST - 16, 16)) + (HIST - 16,)

    def w_body(s, _):
        for c in chunk_offs:
            iv = idx_v[s, pl.ds(c, 16)]
            rv = rat_v[s, pl.ds(c, 16)]
            w_v[s, pl.ds(c, 16)] = jnp.where(iv == pad_idx, 0.0,
                                             jnp.abs(rv))
        return 0
    lax.fori_loop(0, BPW, w_body, 0)

    def issue(s, buf, sem):
        pltpu.async_copy(table.at[idx_v.at[s, pl.ds(0, SPL)]],
                         buf.at[pl.ds(0, SPL)], sem)
        pltpu.async_copy(table.at[idx_v.at[s, pl.ds(SPL, SPL2)]],
                         buf.at[pl.ds(SPL, SPL2)], sem)

    def wait(s, buf, sem):
        pltpu.make_async_copy(table.at[idx_v.at[s, pl.ds(0, SPL)]],
                              buf.at[pl.ds(0, SPL)], sem).wait()
        pltpu.make_async_copy(table.at[idx_v.at[s, pl.ds(SPL, SPL2)]],
                              buf.at[pl.ds(SPL, SPL2)], sem).wait()

    zero16 = jnp.zeros((16,), jnp.float32)

    def compute(s, buf):
        def rows16(a0, a1, a2, wsum, wv, g, n, lane_off=0):
            for k in range(n):
                h = g * 16 + k
                w = wv[k + lane_off]
                wsum[k % 4] = wsum[k % 4] + w
                r0 = buf[h, pl.ds(0, 16)]
                r1 = buf[h, pl.ds(16, 16)]
                r2 = buf[h, pl.ds(24, 16)]
                a0 = a0 + w * r0
                a1 = a1 + w * r1
                a2 = a2 + w * r2
            return a0, a1, a2

        def grp_body(g, carry):
            a0, a1, a2, s0, s1, s2, s3 = carry
            wsum = [s0, s1, s2, s3]
            wv = w_v[s, pl.ds(g * 16, 16)]
            a0, a1, a2 = rows16(a0, a1, a2, wsum, wv, g, 16)
            return (a0, a1, a2, *wsum)

        z = jnp.float32(0.0)
        a0, a1, a2, s0, s1, s2, s3 = lax.fori_loop(
            0, (HIST // 16), grp_body, (zero16, zero16, zero16, z, z, z, z))
        wsum = [s0, s1, s2, s3]
        wv = w_v[s, pl.ds(HIST - 16, 16)]
        a0, a1, a2 = rows16(a0, a1, a2, wsum, wv, 12, 8, lane_off=8)
        ws = (wsum[0] + wsum[1]) + (wsum[2] + wsum[3])
        wsb = jnp.broadcast_to(ws, (16,))
        inv = 1.0 / jnp.maximum(wsb, 1e-6)
        ob = s * DP
        out_v[pl.ds(ob + 24, 16)] = a2 * inv
        out_v[pl.ds(ob + 16, 16)] = a1 * inv
        out_v[pl.ds(ob, 16)] = a0 * inv
        return ws

    issue(0, rows_a, sem_a)

    def chunk(g, _):
        s = g * 2
        issue(s + 1, rows_b, sem_b)
        wait(s, rows_a, sem_a)
        compute(s, rows_a)
        s2 = jnp.minimum(s + 2, BPW - 1)
        issue(s2, rows_a, sem_a)
        wait(s + 1, rows_b, sem_b)
        compute(s + 1, rows_b)
        return _

    lax.fori_loop(0, BPW // 2, chunk, 0)
    wait(BPW - 1, rows_a, sem_a)
    pltpu.make_async_copy(table.at[tidx_v], trows_v, sem_t).wait()
    pltpu.make_async_copy(ytab.at[yidx_v], yrows_v, sem_y).wait()
    pltpu.make_async_copy(tstab.at[tsidx_v], tsrows_v, sem_ts).wait()

    pltpu.sync_copy(out_v, pool_out.at[pl.ds(base * DP, BPW * DP)])
    pltpu.sync_copy(trows_v, trows_out.at[pl.ds(base, BPW)])
    pltpu.sync_copy(yrows_v, yrows_out.at[pl.ds(base, BPW)])
    pltpu.sync_copy(tsrows_v, tsrows_out.at[pl.ds(base, BPW)])


@functools.lru_cache(maxsize=1)
def _sc_pool_call():
    return pl.kernel(
        _sc_pool,
        out_type=(
            jax.ShapeDtypeStruct((B * DP,), jnp.float32),
            jax.ShapeDtypeStruct((B, D), jnp.float32),
            jax.ShapeDtypeStruct((B, TP), jnp.float32),
            jax.ShapeDtypeStruct((B, TP), jnp.float32),
        ),
        mesh=plsc.VectorSubcoreMesh(
            core_axis_name="c", subcore_axis_name="s",
            num_cores=NC, num_subcores=NS),
        compiler_params=pltpu.CompilerParams(use_tc_tiling_on_sc=False),
        scratch_types=[
            pltpu.VMEM((BPW, HIST), jnp.int32),
            pltpu.VMEM((BPW, HIST), jnp.float32),
            pltpu.VMEM((BPW, HIST), jnp.float32),
            pltpu.VMEM((BPW,), jnp.int32),
            pltpu.VMEM((BPW,), jnp.int32),
            pltpu.VMEM((BPW,), jnp.int32),
            pltpu.VMEM((HIST, D), jnp.float32),
            pltpu.VMEM((HIST, D), jnp.float32),
            pltpu.VMEM((BPW * DP,), jnp.float32),
            pltpu.VMEM((BPW, D), jnp.float32),
            pltpu.VMEM((BPW, TP), jnp.float32),
            pltpu.VMEM((BPW, TP), jnp.float32),
            pltpu.SemaphoreType.DMA,
            pltpu.SemaphoreType.DMA,
            pltpu.SemaphoreType.DMA,
            pltpu.SemaphoreType.DMA,
            pltpu.SemaphoreType.DMA,
        ],
    )


BLK = 512
NBLK = B // BLK


def _tc_pre(genres_ref, tags_ref, genome_ref, ugc_ref,
            w_ig, b_ig, w_it, b_it, w_igt, b_igt, w_ug, b_ug, out_ref):
    f32 = jnp.float32

    def mm(a, b):
        return jnp.dot(a, b, preferred_element_type=f32)

    ig = jnp.tanh(mm(genres_ref[...], w_ig[...]) + b_ig[...])
    it = jnp.tanh(mm(tags_ref[...], w_it[...]) + b_it[...])
    igt = jnp.tanh(mm(genome_ref[...], w_igt[...]) + b_igt[...])
    ug = jnp.tanh(mm(ugc_ref[...], w_ug[...]) + b_ug[...])
    out_ref[...] = jnp.concatenate([ig, it, igt, ug], axis=1)


def _tc_post(pre_ref, hist_ref, trow_ref, yrow_ref, tsrow_ref,
             w_item, b_item, w_y, b_y, w_ts, b_ts, out_ref):
    f32 = jnp.float32

    def mm(a, b):
        return jnp.dot(a, b, preferred_element_type=f32)

    item = jnp.tanh(mm(trow_ref[...], w_item[...]) + b_item[...])
    yemb = jnp.tanh(mm(yrow_ref[...], w_y[...]) + b_y[...])
    tsemb = jnp.tanh(mm(tsrow_ref[...], w_ts[...]) + b_ts[...])

    pre = pre_ref[...]
    ig = pre[:, 0:10]
    it = pre[:, 10:30]
    igt = pre[:, 30:50]
    ug = pre[:, 50:100]
    hist = hist_ref[:, :D]

    u = jnp.concatenate([hist, ug, tsemb], axis=1)
    v = jnp.concatenate([ig, it, igt, item, yemb], axis=1)
    out_ref[...] = jnp.sum(u * v, axis=1, keepdims=True)


def _row_spec(cols):
    return pl.BlockSpec((BLK, cols), lambda i: (i, 0))


def _full_spec(shape):
    nd = len(shape)
    return pl.BlockSpec(shape, lambda i: (0,) * nd)


def kernel(user_genre_contexts, user_tag_contexts, user_watch_history,
           user_watch_history_ratings, timestamps, movie_genres, movie_tags,
           movie_genome_tags, years, target_movieId, item_table, W_item,
           b_item, W_ig, b_ig, W_it, b_it, W_igt, b_igt, year_table, W_y,
           b_y, W_ug, b_ug, ts_table, W_ts, b_ts):
    idx2 = user_watch_history.astype(jnp.int32)
    tidx = target_movieId.astype(jnp.int32)
    yidx = years.astype(jnp.int32)
    tsidx = timestamps.astype(jnp.int32)

    td = year_table.shape[1]
    ytab_p = jnp.pad(year_table, ((0, 0), (0, TP - td)))
    tstab_p = jnp.pad(ts_table, ((0, 0), (0, TP - td)))
    w_y_p = jnp.pad(W_y, ((0, TP - td), (0, 0)))
    w_ts_p = jnp.pad(W_ts, ((0, TP - td), (0, 0)))

    pool_flat, trows, yrows, tsrows = _sc_pool_call()(
        item_table, idx2, user_watch_history_ratings, tidx,
        ytab_p, tstab_p, yidx, tsidx)
    hist_pool = pool_flat.reshape(B, DP)

    b2 = lambda x: x.reshape(1, -1)

    pre = pl.pallas_call(
        _tc_pre,
        grid=(NBLK,),
        in_specs=[
            _row_spec(movie_genres.shape[1]),
            _row_spec(movie_tags.shape[1]),
            _row_spec(movie_genome_tags.shape[1]),
            _row_spec(user_genre_contexts.shape[1]),
            _full_spec(W_ig.shape), _full_spec((1, b_ig.shape[0])),
            _full_spec(W_it.shape), _full_spec((1, b_it.shape[0])),
            _full_spec(W_igt.shape), _full_spec((1, b_igt.shape[0])),
            _full_spec(W_ug.shape), _full_spec((1, b_ug.shape[0])),
        ],
        out_specs=pl.BlockSpec((BLK, 100), lambda i: (i, 0)),
        out_shape=jax.ShapeDtypeStruct((B, 100), jnp.float32),
    )(movie_genres, movie_tags, movie_genome_tags, user_genre_contexts,
      W_ig, b2(b_ig), W_it, b2(b_it), W_igt, b2(b_igt), W_ug, b2(b_ug))

    out = pl.pallas_call(
        _tc_post,
        grid=(NBLK,),
        in_specs=[
            _row_spec(100),
            _row_spec(DP),
            _row_spec(D),
            _row_spec(TP),
            _row_spec(TP),
            _full_spec(W_item.shape), _full_spec((1, b_item.shape[0])),
            _full_spec(w_y_p.shape), _full_spec((1, b_y.shape[0])),
            _full_spec(w_ts_p.shape), _full_spec((1, b_ts.shape[0])),
        ],
        out_specs=pl.BlockSpec((BLK, 1), lambda i: (i, 0)),
        out_shape=jax.ShapeDtypeStruct((B, 1), jnp.float32),
    )(pre, hist_pool, trows, yrows, tsrows,
      W_item, b2(b_item), w_y_p, b2(b_y), w_ts_p, b2(b_ts))

    return out.reshape(B)


# speedup vs baseline: 12.7616x; 1.0014x over previous
import functools

import jax
import jax.numpy as jnp
from jax import lax
from jax.experimental import pallas as pl
from jax.experimental.pallas import tpu as pltpu
from jax.experimental.pallas import tpu_sc as plsc

B = 4096
HIST = 200
D = 40
DP = 48
TP = 16
NC = 2
NS = 16
NW = NC * NS
BPW = B // NW
SPL = 104
SPL2 = HIST - SPL


def _sc_pool(table, idx2, rat2, tidx, ytab, tstab, yidx, tsidx,
             pool_out, trows_out, yrows_out, tsrows_out,
             idx_v, rat_v, w_v, tidx_v, yidx_v, tsidx_v,
             rows_a, rows_b, out_v, trows_v, yrows_v, tsrows_v,
             sem_a, sem_b, sem_t, sem_y, sem_ts):
    pad_idx = table.shape[0] - 1
    wid = lax.axis_index("s") * NC + lax.axis_index("c")
    base = wid * BPW

    pltpu.sync_copy(idx2.at[pl.ds(base * HIST, BPW * HIST)], idx_v)
    pltpu.sync_copy(rat2.at[pl.ds(base * HIST, BPW * HIST)], rat_v)
    pltpu.sync_copy(tidx.at[pl.ds(base, BPW)], tidx_v)
    pltpu.sync_copy(yidx.at[pl.ds(base, BPW)], yidx_v)
    pltpu.sync_copy(tsidx.at[pl.ds(base, BPW)], tsidx_v)
    pltpu.async_copy(table.at[tidx_v], trows_v, sem_t)
    pltpu.async_copy(ytab.at[yidx_v], yrows_v, sem_y)
    pltpu.async_copy(tstab.at[tsidx_v], tsrows_v, sem_ts)

    chunk_offs = tuple(range(0, HIST - 16, 16)) + (HIST - 16,)

    def w_body(s, _):
        sb = s * HIST
        for c in chunk_offs:
            iv = idx_v[pl.ds(sb + c, 16)]
            rv = rat_v[pl.ds(sb + c, 16)]
            w_v[pl.ds(sb + c, 16)] = jnp.where(iv == pad_idx, 0.0,
                                               jnp.abs(rv))
        return 0
    lax.fori_loop(0, BPW, w_body, 0)

    def issue(s, buf, sem):
        pltpu.async_copy(table.at[idx_v.at[pl.ds(s * HIST, SPL)]],
                         buf.at[pl.ds(0, SPL)], sem)
        pltpu.async_copy(table.at[idx_v.at[pl.ds(s * HIST + SPL, SPL2)]],
                         buf.at[pl.ds(SPL, SPL2)], sem)

    def wait(s, buf, sem):
        pltpu.make_async_copy(table.at[idx_v.at[pl.ds(s * HIST, SPL)]],
                              buf.at[pl.ds(0, SPL)], sem).wait()
        pltpu.make_async_copy(table.at[idx_v.at[pl.ds(s * HIST + SPL, SPL2)]],
                              buf.at[pl.ds(SPL, SPL2)], sem).wait()

    zero16 = jnp.zeros((16,), jnp.float32)

    def compute(s, buf):
        def rows16(a0, a1, a2, wsum, wv, g, n, lane_off=0):
            for k in range(n):
                h = g * 16 + k
                w = wv[k + lane_off]
                wsum[k % 4] = wsum[k % 4] + w
                r0 = buf[h, pl.ds(0, 16)]
                r1 = buf[h, pl.ds(16, 16)]
                r2 = buf[h, pl.ds(24, 16)]
                a0 = a0 + w * r0
                a1 = a1 + w * r1
                a2 = a2 + w * r2
            return a0, a1, a2

        def grp_body(g, carry):
            a0, a1, a2, s0, s1, s2, s3 = carry
            wsum = [s0, s1, s2, s3]
            wv = w_v[pl.ds(s * HIST + g * 16, 16)]
            a0, a1, a2 = rows16(a0, a1, a2, wsum, wv, g, 16)
            return (a0, a1, a2, *wsum)

        z = jnp.float32(0.0)
        a0, a1, a2, s0, s1, s2, s3 = lax.fori_loop(
            0, (HIST // 16), grp_body, (zero16, zero16, zero16, z, z, z, z))
        wsum = [s0, s1, s2, s3]
        wv = w_v[pl.ds(s * HIST + HIST - 16, 16)]
        a0, a1, a2 = rows16(a0, a1, a2, wsum, wv, 12, 8, lane_off=8)
        ws = (wsum[0] + wsum[1]) + (wsum[2] + wsum[3])
        wsb = jnp.broadcast_to(ws, (16,))
        inv = 1.0 / jnp.maximum(wsb, 1e-6)
        ob = s * DP
        out_v[pl.ds(ob + 24, 16)] = a2 * inv
        out_v[pl.ds(ob + 16, 16)] = a1 * inv
        out_v[pl.ds(ob, 16)] = a0 * inv
        return ws

    issue(0, rows_a, sem_a)

    def chunk(g, _):
        s = g * 2
        issue(s + 1, rows_b, sem_b)
        wait(s, rows_a, sem_a)
        compute(s, rows_a)
        s2 = jnp.minimum(s + 2, BPW - 1)
        issue(s2, rows_a, sem_a)
        wait(s + 1, rows_b, sem_b)
        compute(s + 1, rows_b)
        return _

    lax.fori_loop(0, BPW // 2, chunk, 0)
    wait(BPW - 1, rows_a, sem_a)
    pltpu.make_async_copy(table.at[tidx_v], trows_v, sem_t).wait()
    pltpu.make_async_copy(ytab.at[yidx_v], yrows_v, sem_y).wait()
    pltpu.make_async_copy(tstab.at[tsidx_v], tsrows_v, sem_ts).wait()

    pltpu.sync_copy(out_v, pool_out.at[pl.ds(base * DP, BPW * DP)])
    pltpu.sync_copy(trows_v, trows_out.at[pl.ds(base, BPW)])
    pltpu.sync_copy(yrows_v, yrows_out.at[pl.ds(base, BPW)])
    pltpu.sync_copy(tsrows_v, tsrows_out.at[pl.ds(base, BPW)])


@functools.lru_cache(maxsize=1)
def _sc_pool_call():
    return pl.kernel(
        _sc_pool,
        out_type=(
            jax.ShapeDtypeStruct((B * DP,), jnp.float32),
            jax.ShapeDtypeStruct((B, D), jnp.float32),
            jax.ShapeDtypeStruct((B, TP), jnp.float32),
            jax.ShapeDtypeStruct((B, TP), jnp.float32),
        ),
        mesh=plsc.VectorSubcoreMesh(
            core_axis_name="c", subcore_axis_name="s",
            num_cores=NC, num_subcores=NS),
        compiler_params=pltpu.CompilerParams(use_tc_tiling_on_sc=False),
        scratch_types=[
            pltpu.VMEM((BPW * HIST,), jnp.int32),
            pltpu.VMEM((BPW * HIST,), jnp.float32),
            pltpu.VMEM((BPW * HIST,), jnp.float32),
            pltpu.VMEM((BPW,), jnp.int32),
            pltpu.VMEM((BPW,), jnp.int32),
            pltpu.VMEM((BPW,), jnp.int32),
            pltpu.VMEM((HIST, D), jnp.float32),
            pltpu.VMEM((HIST, D), jnp.float32),
            pltpu.VMEM((BPW * DP,), jnp.float32),
            pltpu.VMEM((BPW, D), jnp.float32),
            pltpu.VMEM((BPW, TP), jnp.float32),
            pltpu.VMEM((BPW, TP), jnp.float32),
            pltpu.SemaphoreType.DMA,
            pltpu.SemaphoreType.DMA,
            pltpu.SemaphoreType.DMA,
            pltpu.SemaphoreType.DMA,
            pltpu.SemaphoreType.DMA,
        ],
    )


BLK = 512
NBLK = B // BLK


def _tc_pre(genres_ref, tags_ref, genome_ref, ugc_ref,
            w_ig, b_ig, w_it, b_it, w_igt, b_igt, w_ug, b_ug, out_ref):
    f32 = jnp.float32

    def mm(a, b):
        return jnp.dot(a, b, preferred_element_type=f32)

    ig = jnp.tanh(mm(genres_ref[...], w_ig[...]) + b_ig[...])
    it = jnp.tanh(mm(tags_ref[...], w_it[...]) + b_it[...])
    igt = jnp.tanh(mm(genome_ref[...], w_igt[...]) + b_igt[...])
    ug = jnp.tanh(mm(ugc_ref[...], w_ug[...]) + b_ug[...])
    out_ref[...] = jnp.concatenate([ig, it, igt, ug], axis=1)


def _tc_post(pre_ref, hist_ref, trow_ref, yrow_ref, tsrow_ref,
             w_item, b_item, w_y, b_y, w_ts, b_ts, out_ref):
    f32 = jnp.float32

    def mm(a, b):
        return jnp.dot(a, b, preferred_element_type=f32)

    item = jnp.tanh(mm(trow_ref[...], w_item[...]) + b_item[...])
    yemb = jnp.tanh(mm(yrow_ref[...], w_y[...]) + b_y[...])
    tsemb = jnp.tanh(mm(tsrow_ref[...], w_ts[...]) + b_ts[...])

    pre = pre_ref[...]
    ig = pre[:, 0:10]
    it = pre[:, 10:30]
    igt = pre[:, 30:50]
    ug = pre[:, 50:100]
    hist = hist_ref[:, :D]

    u = jnp.concatenate([hist, ug, tsemb], axis=1)
    v = jnp.concatenate([ig, it, igt, item, yemb], axis=1)
    out_ref[...] = jnp.sum(u * v, axis=1, keepdims=True)


def _row_spec(cols):
    return pl.BlockSpec((BLK, cols), lambda i: (i, 0))


def _full_spec(shape):
    nd = len(shape)
    return pl.BlockSpec(shape, lambda i: (0,) * nd)


def kernel(user_genre_contexts, user_tag_contexts, user_watch_history,
           user_watch_history_ratings, timestamps, movie_genres, movie_tags,
           movie_genome_tags, years, target_movieId, item_table, W_item,
           b_item, W_ig, b_ig, W_it, b_it, W_igt, b_igt, year_table, W_y,
           b_y, W_ug, b_ug, ts_table, W_ts, b_ts):
    idx2 = user_watch_history.reshape(-1).astype(jnp.int32)
    rat2 = user_watch_history_ratings.reshape(-1)
    tidx = target_movieId.astype(jnp.int32)
    yidx = years.astype(jnp.int32)
    tsidx = timestamps.astype(jnp.int32)

    td = year_table.shape[1]
    ytab_p = jnp.pad(year_table, ((0, 0), (0, TP - td)))
    tstab_p = jnp.pad(ts_table, ((0, 0), (0, TP - td)))
    w_y_p = jnp.pad(W_y, ((0, TP - td), (0, 0)))
    w_ts_p = jnp.pad(W_ts, ((0, TP - td), (0, 0)))

    pool_flat, trows, yrows, tsrows = _sc_pool_call()(
        item_table, idx2, rat2, tidx,
        ytab_p, tstab_p, yidx, tsidx)
    hist_pool = pool_flat.reshape(B, DP)

    b2 = lambda x: x.reshape(1, -1)

    pre = pl.pallas_call(
        _tc_pre,
        grid=(NBLK,),
        in_specs=[
            _row_spec(movie_genres.shape[1]),
            _row_spec(movie_tags.shape[1]),
            _row_spec(movie_genome_tags.shape[1]),
            _row_spec(user_genre_contexts.shape[1]),
            _full_spec(W_ig.shape), _full_spec((1, b_ig.shape[0])),
            _full_spec(W_it.shape), _full_spec((1, b_it.shape[0])),
            _full_spec(W_igt.shape), _full_spec((1, b_igt.shape[0])),
            _full_spec(W_ug.shape), _full_spec((1, b_ug.shape[0])),
        ],
        out_specs=pl.BlockSpec((BLK, 100), lambda i: (i, 0)),
        out_shape=jax.ShapeDtypeStruct((B, 100), jnp.float32),
    )(movie_genres, movie_tags, movie_genome_tags, user_genre_contexts,
      W_ig, b2(b_ig), W_it, b2(b_it), W_igt, b2(b_igt), W_ug, b2(b_ug))

    out = pl.pallas_call(
        _tc_post,
        grid=(NBLK,),
        in_specs=[
            _row_spec(100),
            _row_spec(DP),
            _row_spec(D),
            _row_spec(TP),
            _row_spec(TP),
            _full_spec(W_item.shape), _full_spec((1, b_item.shape[0])),
            _full_spec(w_y_p.shape), _full_spec((1, b_y.shape[0])),
            _full_spec(w_ts_p.shape), _full_spec((1, b_ts.shape[0])),
        ],
        out_specs=pl.BlockSpec((BLK, 1), lambda i: (i, 0)),
        out_shape=jax.ShapeDtypeStruct((B, 1), jnp.float32),
    )(pre, hist_pool, trows, yrows, tsrows,
      W_item, b2(b_item), w_y_p, b2(b_y), w_ts_p, b2(b_ts))

    return out.reshape(B)
